# baseline (speedup 1.0000x reference)
import functools

import jax
import jax.numpy as jnp
from jax import lax
from jax.experimental import pallas as pl
from jax.experimental.pallas import tpu as pltpu
from jax.experimental.pallas import tpu_sc as plsc

_N = 10000
_E = 320000
_D = 128
_R = 8

_NC = 2
_NS = 16
_NW = _NC * _NS
_CH = 128
_CPT = 80
_CPH = 20
_NBUF = 2
_EPAD = _NW * _CPT * _CH
_NPAD = 10240
_RPT = _NPAD // _NS

_BN = 1000
_NB = _N // _BN


def _transform1(x, W, lw, b2d):
    def body(x_ref, w_ref, lw_ref, b_ref, xw_ref, s_ref):
        xb = x_ref[...].astype(jnp.bfloat16)
        wb = w_ref[0].astype(jnp.bfloat16)
        xw_ref[0] = jnp.dot(xb, wb, preferred_element_type=jnp.float32)

        @pl.when(pl.program_id(0) == 0)
        def _():
            lwb = lw_ref[...].astype(jnp.bfloat16)
            s_ref[...] = (
                jnp.dot(xb, lwb, preferred_element_type=jnp.float32)
                + b_ref[...]
            )

    return pl.pallas_call(
        body,
        grid=(_R,),
        in_specs=[
            pl.BlockSpec((_N, _D), lambda r: (0, 0)),
            pl.BlockSpec((1, _D, _D), lambda r: (r, 0, 0)),
            pl.BlockSpec((_D, _D), lambda r: (0, 0)),
            pl.BlockSpec((1, _D), lambda r: (0, 0)),
        ],
        out_specs=[
            pl.BlockSpec((1, _N, _D), lambda r: (r, 0, 0)),
            pl.BlockSpec((_N, _D), lambda r: (0, 0)),
        ],
        out_shape=[
            jax.ShapeDtypeStruct((_R, _N, _D), jnp.float32),
            jax.ShapeDtypeStruct((_N, _D), jnp.float32),
        ],
    )(x, W, lw, b2d)


def _transform2(p, s1, W, lw, b2d):
    def body(p_ref, s1_ref, w_ref, lw_ref, b_ref, xw_ref, s_ref, h_ref):
        @pl.when(pl.program_id(0) == 0)
        def _():
            h_ref[...] = jnp.maximum(
                p_ref[0] + p_ref[1] + s1_ref[...], 0.0
            ).astype(jnp.bfloat16)

        h = h_ref[...]
        wb = w_ref[0].astype(jnp.bfloat16)
        xw_ref[0] = jnp.dot(h, wb, preferred_element_type=jnp.float32)

        @pl.when(pl.program_id(0) == 0)
        def _():
            lwb = lw_ref[...].astype(jnp.bfloat16)
            s_ref[...] = (
                jnp.dot(h, lwb, preferred_element_type=jnp.float32)
                + b_ref[...]
            )

    return pl.pallas_call(
        body,
        grid=(_R,),
        in_specs=[
            pl.BlockSpec((2, _N, _D), lambda r: (0, 0, 0)),
            pl.BlockSpec((_N, _D), lambda r: (0, 0)),
            pl.BlockSpec((1, _D, _D), lambda r: (r, 0, 0)),
            pl.BlockSpec((_D, _D), lambda r: (0, 0)),
            pl.BlockSpec((1, _D), lambda r: (0, 0)),
        ],
        out_specs=[
            pl.BlockSpec((1, _N, _D), lambda r: (r, 0, 0)),
            pl.BlockSpec((_N, _D), lambda r: (0, 0)),
        ],
        out_shape=[
            jax.ShapeDtypeStruct((_R, _N, _D), jnp.float32),
            jax.ShapeDtypeStruct((_N, _D), jnp.float32),
        ],
        scratch_shapes=[pltpu.VMEM((_N, _D), jnp.bfloat16)],
    )(p, s1, W, lw, b2d)


def _combine(p, s2):
    def body(p_ref, s_ref, o_ref):
        o_ref[...] = p_ref[0] + p_ref[1] + s_ref[...]

    return pl.pallas_call(
        body,
        grid=(1,),
        in_specs=[
            pl.BlockSpec((2, _N, _D), lambda i: (0, 0, 0)),
            pl.BlockSpec((_N, _D), lambda i: (0, 0)),
        ],
        out_specs=pl.BlockSpec((_N, _D), lambda i: (0, 0)),
        out_shape=jax.ShapeDtypeStruct((_N, _D), jnp.float32),
    )(p, s2)


def _sc_aggregate(table, gidx, didx, zeros):
    mesh = plsc.VectorSubcoreMesh(core_axis_name="c", subcore_axis_name="s")

    @functools.partial(
        pl.kernel,
        mesh=mesh,
        out_type=jax.ShapeDtypeStruct((_NC, _NPAD, _D), jnp.float32),
        scratch_types=[
            pltpu.VMEM((_CPH, _CH), jnp.int32),
            pltpu.VMEM((_CPH, _CH), jnp.int32),
            pltpu.VMEM((_CH, _D), jnp.float32),
            pltpu.VMEM((_CH, _D), jnp.float32),
            pltpu.VMEM((16, _D), jnp.float32),
            pltpu.VMEM_SHARED((_NPAD, _D), jnp.float32),
            pltpu.SemaphoreType.DMA,
            pltpu.SemaphoreType.DMA,
        ],
    )
    def k(table_hbm, gidx_hbm, didx_hbm, z_hbm, out_hbm,
          gidx_v, didx_v, rows_a, rows_b, z_v, acc,
          sem_a, sem_b):
        c = lax.axis_index("c")
        s = lax.axis_index("s")
        wid = c * _NS + s

        pltpu.sync_copy(z_hbm, z_v)

        @pl.loop(0, _RPT // 16)
        def _(k2):
            pltpu.sync_copy(z_v, acc.at[pl.ds(s * _RPT + k2 * 16, 16)])

        plsc.subcore_barrier()

        bufs = ((rows_a, sem_a), (rows_b, sem_b))

        @pl.loop(0, _CPT // _CPH)
        def _(ph):
            pltpu.sync_copy(gidx_hbm.at[wid, ph], gidx_v)
            pltpu.sync_copy(didx_hbm.at[wid, ph], didx_v)
            for b, (rows, sem) in enumerate(bufs):
                pltpu.async_copy(table_hbm.at[gidx_v.at[b]], rows, sem)

            @pl.loop(0, _CPH // _NBUF)
            def _(jj):
                j = jj * _NBUF

                def step(off, rows, sem):
                    pltpu.make_async_copy(
                        table_hbm.at[gidx_v.at[j + off]], rows, sem
                    ).wait()
                    pltpu.sync_copy(rows, acc.at[didx_v.at[j + off]], add=True)

                    @pl.when(j + off + _NBUF < _CPH)
                    def _():
                        pltpu.async_copy(
                            table_hbm.at[gidx_v.at[j + off + _NBUF]], rows, sem
                        )

                for b, (rows, sem) in enumerate(bufs):
                    step(b, rows, sem)

        plsc.subcore_barrier()

        pltpu.sync_copy(
            acc.at[pl.ds(s * _RPT, _RPT)],
            out_hbm.at[c, pl.ds(s * _RPT, _RPT)],
        )

    return k(table, gidx, didx, zeros)


def kernel(inputs, edge_index, etypes, W1, loop1, b1, W2, loop2, b2):
    x = inputs
    src = edge_index[0].astype(jnp.int32)
    dst = edge_index[1].astype(jnp.int32)
    et = etypes.astype(jnp.int32)

    pad = _EPAD - src.shape[0]
    pad_g = jnp.arange(pad, dtype=jnp.int32) % (_R * _N)
    pad_d = _N + jnp.arange(pad, dtype=jnp.int32) % (_NPAD - _N)
    idx_shape = (_NW, _CPT // _CPH, _CPH, _CH)
    gidx = jnp.concatenate([et * _N + src, pad_g]).reshape(idx_shape)
    didx = jnp.concatenate([dst, pad_d]).reshape(idx_shape)
    zeros = jnp.zeros((16, _D), jnp.float32)
    b1r = b1.reshape(1, _D)
    b2r = b2.reshape(1, _D)

    xw1, self1 = _transform1(x, W1, loop1, b1r)
    p1 = _sc_aggregate(xw1.reshape(_R * _N, _D), gidx, didx, zeros)
    xw2, self2 = _transform2(p1, self1, W2, loop2, b2r)
    p2 = _sc_aggregate(xw2.reshape(_R * _N, _D), gidx, didx, zeros)
    return _combine(p2, self2)


# speedup vs baseline: 1.0170x; 1.0170x over previous
import functools

import jax
import jax.numpy as jnp
from jax import lax
from jax.experimental import pallas as pl
from jax.experimental.pallas import tpu as pltpu
from jax.experimental.pallas import tpu_sc as plsc

_N = 10000
_E = 320000
_D = 128
_R = 8

_NC = 2
_NS = 16
_NW = _NC * _NS
_CH = 96
_CPT = 108
_CPH = 18
_NBUF = 3
_EPAD = _NW * _CPT * _CH
_NPAD = 10240
_RPT = _NPAD // _NS

_BN = 1000
_NB = _N // _BN


def _transform1(x, W, lw, b2d):
    def body(x_ref, w_ref, lw_ref, b_ref, xw_ref, s_ref):
        xb = x_ref[...].astype(jnp.bfloat16)
        wb = w_ref[0].astype(jnp.bfloat16)
        xw_ref[0] = jnp.dot(xb, wb, preferred_element_type=jnp.float32)

        @pl.when(pl.program_id(0) == 0)
        def _():
            lwb = lw_ref[...].astype(jnp.bfloat16)
            s_ref[...] = (
                jnp.dot(xb, lwb, preferred_element_type=jnp.float32)
                + b_ref[...]
            )

    return pl.pallas_call(
        body,
        grid=(_R,),
        in_specs=[
            pl.BlockSpec((_N, _D), lambda r: (0, 0)),
            pl.BlockSpec((1, _D, _D), lambda r: (r, 0, 0)),
            pl.BlockSpec((_D, _D), lambda r: (0, 0)),
            pl.BlockSpec((1, _D), lambda r: (0, 0)),
        ],
        out_specs=[
            pl.BlockSpec((1, _N, _D), lambda r: (r, 0, 0)),
            pl.BlockSpec((_N, _D), lambda r: (0, 0)),
        ],
        out_shape=[
            jax.ShapeDtypeStruct((_R, _N, _D), jnp.float32),
            jax.ShapeDtypeStruct((_N, _D), jnp.float32),
        ],
    )(x, W, lw, b2d)


def _transform2(p, s1, W, lw, b2d):
    def body(p_ref, s1_ref, w_ref, lw_ref, b_ref, xw_ref, s_ref, h_ref):
        @pl.when(pl.program_id(0) == 0)
        def _():
            h_ref[...] = jnp.maximum(
                p_ref[0] + p_ref[1] + s1_ref[...], 0.0
            ).astype(jnp.bfloat16)

        h = h_ref[...]
        wb = w_ref[0].astype(jnp.bfloat16)
        xw_ref[0] = jnp.dot(h, wb, preferred_element_type=jnp.float32)

        @pl.when(pl.program_id(0) == 0)
        def _():
            lwb = lw_ref[...].astype(jnp.bfloat16)
            s_ref[...] = (
                jnp.dot(h, lwb, preferred_element_type=jnp.float32)
                + b_ref[...]
            )

    return pl.pallas_call(
        body,
        grid=(_R,),
        in_specs=[
            pl.BlockSpec((2, _N, _D), lambda r: (0, 0, 0)),
            pl.BlockSpec((_N, _D), lambda r: (0, 0)),
            pl.BlockSpec((1, _D, _D), lambda r: (r, 0, 0)),
            pl.BlockSpec((_D, _D), lambda r: (0, 0)),
            pl.BlockSpec((1, _D), lambda r: (0, 0)),
        ],
        out_specs=[
            pl.BlockSpec((1, _N, _D), lambda r: (r, 0, 0)),
            pl.BlockSpec((_N, _D), lambda r: (0, 0)),
        ],
        out_shape=[
            jax.ShapeDtypeStruct((_R, _N, _D), jnp.float32),
            jax.ShapeDtypeStruct((_N, _D), jnp.float32),
        ],
        scratch_shapes=[pltpu.VMEM((_N, _D), jnp.bfloat16)],
    )(p, s1, W, lw, b2d)


def _combine(p, s2):
    def body(p_ref, s_ref, o_ref):
        o_ref[...] = p_ref[0] + p_ref[1] + s_ref[...]

    return pl.pallas_call(
        body,
        grid=(1,),
        in_specs=[
            pl.BlockSpec((2, _N, _D), lambda i: (0, 0, 0)),
            pl.BlockSpec((_N, _D), lambda i: (0, 0)),
        ],
        out_specs=pl.BlockSpec((_N, _D), lambda i: (0, 0)),
        out_shape=jax.ShapeDtypeStruct((_N, _D), jnp.float32),
    )(p, s2)


def _sc_aggregate(table, gidx, didx, zeros):
    mesh = plsc.VectorSubcoreMesh(core_axis_name="c", subcore_axis_name="s")

    @functools.partial(
        pl.kernel,
        mesh=mesh,
        out_type=jax.ShapeDtypeStruct((_NC, _NPAD, _D), jnp.float32),
        scratch_types=[
            pltpu.VMEM((_CPH, _CH), jnp.int32),
            pltpu.VMEM((_CPH, _CH), jnp.int32),
            pltpu.VMEM((_CH, _D), jnp.float32),
            pltpu.VMEM((_CH, _D), jnp.float32),
            pltpu.VMEM((_CH, _D), jnp.float32),
            pltpu.VMEM((16, _D), jnp.float32),
            pltpu.VMEM_SHARED((_NPAD, _D), jnp.float32),
            pltpu.SemaphoreType.DMA,
            pltpu.SemaphoreType.DMA,
            pltpu.SemaphoreType.DMA,
        ],
    )
    def k(table_hbm, gidx_hbm, didx_hbm, z_hbm, out_hbm,
          gidx_v, didx_v, rows_a, rows_b, rows_c, z_v, acc,
          sem_a, sem_b, sem_c):
        c = lax.axis_index("c")
        s = lax.axis_index("s")
        wid = c * _NS + s

        pltpu.sync_copy(z_hbm, z_v)

        @pl.loop(0, _RPT // 16)
        def _(k2):
            pltpu.sync_copy(z_v, acc.at[pl.ds(s * _RPT + k2 * 16, 16)])

        plsc.subcore_barrier()

        bufs = ((rows_a, sem_a), (rows_b, sem_b), (rows_c, sem_c))

        @pl.loop(0, _CPT // _CPH)
        def _(ph):
            pltpu.sync_copy(gidx_hbm.at[wid, ph], gidx_v)
            pltpu.sync_copy(didx_hbm.at[wid, ph], didx_v)
            for b, (rows, sem) in enumerate(bufs):
                pltpu.async_copy(table_hbm.at[gidx_v.at[b]], rows, sem)

            @pl.loop(0, _CPH // _NBUF)
            def _(jj):
                j = jj * _NBUF

                def step(off, rows, sem):
                    pltpu.make_async_copy(
                        table_hbm.at[gidx_v.at[j + off]], rows, sem
                    ).wait()
                    pltpu.sync_copy(rows, acc.at[didx_v.at[j + off]], add=True)

                    @pl.when(j + off + _NBUF < _CPH)
                    def _():
                        pltpu.async_copy(
                            table_hbm.at[gidx_v.at[j + off + _NBUF]], rows, sem
                        )

                for b, (rows, sem) in enumerate(bufs):
                    step(b, rows, sem)

        plsc.subcore_barrier()

        pltpu.sync_copy(
            acc.at[pl.ds(s * _RPT, _RPT)],
            out_hbm.at[c, pl.ds(s * _RPT, _RPT)],
        )

    return k(table, gidx, didx, zeros)


def kernel(inputs, edge_index, etypes, W1, loop1, b1, W2, loop2, b2):
    x = inputs
    src = edge_index[0].astype(jnp.int32)
    dst = edge_index[1].astype(jnp.int32)
    et = etypes.astype(jnp.int32)

    pad = _EPAD - src.shape[0]
    pad_g = jnp.arange(pad, dtype=jnp.int32) % (_R * _N)
    pad_d = _N + jnp.arange(pad, dtype=jnp.int32) % (_NPAD - _N)
    idx_shape = (_NW, _CPT // _CPH, _CPH, _CH)
    gidx = jnp.concatenate([et * _N + src, pad_g]).reshape(idx_shape)
    didx = jnp.concatenate([dst, pad_d]).reshape(idx_shape)
    zeros = jnp.zeros((16, _D), jnp.float32)
    b1r = b1.reshape(1, _D)
    b2r = b2.reshape(1, _D)

    xw1, self1 = _transform1(x, W1, loop1, b1r)
    p1 = _sc_aggregate(xw1.reshape(_R * _N, _D), gidx, didx, zeros)
    xw2, self2 = _transform2(p1, self1, W2, loop2, b2r)
    p2 = _sc_aggregate(xw2.reshape(_R * _N, _D), gidx, didx, zeros)
    return _combine(p2, self2)


# speedup vs baseline: 1.0823x; 1.0641x over previous
import functools

import jax
import jax.numpy as jnp
from jax import lax
from jax.experimental import pallas as pl
from jax.experimental.pallas import tpu as pltpu
from jax.experimental.pallas import tpu_sc as plsc

_N = 10000
_E = 320000
_D = 128
_R = 8

_NC = 2
_NS = 16
_NW = _NC * _NS
_CH = 64
_CPT = 160
_CPH = 40
_NBUF = 4
_EPAD = _NW * _CPT * _CH
_NPAD = 10240
_RPT = _NPAD // _NS

_BN = 1000
_NB = _N // _BN


def _transform1(x, W, lw, b2d):
    def body(x_ref, w_ref, lw_ref, b_ref, xw_ref, s_ref):
        xb = x_ref[...].astype(jnp.bfloat16)
        wb = w_ref[0].astype(jnp.bfloat16)
        xw_ref[0] = jnp.dot(xb, wb, preferred_element_type=jnp.float32)

        @pl.when(pl.program_id(0) == 0)
        def _():
            lwb = lw_ref[...].astype(jnp.bfloat16)
            s_ref[...] = (
                jnp.dot(xb, lwb, preferred_element_type=jnp.float32)
                + b_ref[...]
            )

    return pl.pallas_call(
        body,
        grid=(_R,),
        in_specs=[
            pl.BlockSpec((_N, _D), lambda r: (0, 0)),
            pl.BlockSpec((1, _D, _D), lambda r: (r, 0, 0)),
            pl.BlockSpec((_D, _D), lambda r: (0, 0)),
            pl.BlockSpec((1, _D), lambda r: (0, 0)),
        ],
        out_specs=[
            pl.BlockSpec((1, _N, _D), lambda r: (r, 0, 0)),
            pl.BlockSpec((_N, _D), lambda r: (0, 0)),
        ],
        out_shape=[
            jax.ShapeDtypeStruct((_R, _N, _D), jnp.float32),
            jax.ShapeDtypeStruct((_N, _D), jnp.float32),
        ],
    )(x, W, lw, b2d)


def _transform2(p, s1, W, lw, b2d):
    def body(p_ref, s1_ref, w_ref, lw_ref, b_ref, xw_ref, s_ref, h_ref):
        @pl.when(pl.program_id(0) == 0)
        def _():
            h_ref[...] = jnp.maximum(
                p_ref[0] + p_ref[1] + s1_ref[...], 0.0
            ).astype(jnp.bfloat16)

        h = h_ref[...]
        wb = w_ref[0].astype(jnp.bfloat16)
        xw_ref[0] = jnp.dot(h, wb, preferred_element_type=jnp.float32)

        @pl.when(pl.program_id(0) == 0)
        def _():
            lwb = lw_ref[...].astype(jnp.bfloat16)
            s_ref[...] = (
                jnp.dot(h, lwb, preferred_element_type=jnp.float32)
                + b_ref[...]
            )

    return pl.pallas_call(
        body,
        grid=(_R,),
        in_specs=[
            pl.BlockSpec((2, _N, _D), lambda r: (0, 0, 0)),
            pl.BlockSpec((_N, _D), lambda r: (0, 0)),
            pl.BlockSpec((1, _D, _D), lambda r: (r, 0, 0)),
            pl.BlockSpec((_D, _D), lambda r: (0, 0)),
            pl.BlockSpec((1, _D), lambda r: (0, 0)),
        ],
        out_specs=[
            pl.BlockSpec((1, _N, _D), lambda r: (r, 0, 0)),
            pl.BlockSpec((_N, _D), lambda r: (0, 0)),
        ],
        out_shape=[
            jax.ShapeDtypeStruct((_R, _N, _D), jnp.float32),
            jax.ShapeDtypeStruct((_N, _D), jnp.float32),
        ],
        scratch_shapes=[pltpu.VMEM((_N, _D), jnp.bfloat16)],
    )(p, s1, W, lw, b2d)


def _combine(p, s2):
    def body(p_ref, s_ref, o_ref):
        o_ref[...] = p_ref[0] + p_ref[1] + s_ref[...]

    return pl.pallas_call(
        body,
        grid=(1,),
        in_specs=[
            pl.BlockSpec((2, _N, _D), lambda i: (0, 0, 0)),
            pl.BlockSpec((_N, _D), lambda i: (0, 0)),
        ],
        out_specs=pl.BlockSpec((_N, _D), lambda i: (0, 0)),
        out_shape=jax.ShapeDtypeStruct((_N, _D), jnp.float32),
    )(p, s2)


def _sc_aggregate(table, gidx, didx, zeros):
    mesh = plsc.VectorSubcoreMesh(core_axis_name="c", subcore_axis_name="s")

    @functools.partial(
        pl.kernel,
        mesh=mesh,
        out_type=jax.ShapeDtypeStruct((_NC, _NPAD, _D), jnp.float32),
        scratch_types=[
            pltpu.VMEM((_CPH, _CH), jnp.int32),
            pltpu.VMEM((_CPH, _CH), jnp.int32),
            pltpu.VMEM((_CH, _D), jnp.float32),
            pltpu.VMEM((_CH, _D), jnp.float32),
            pltpu.VMEM((_CH, _D), jnp.float32),
            pltpu.VMEM((_CH, _D), jnp.float32),
            pltpu.VMEM((16, _D), jnp.float32),
            pltpu.VMEM_SHARED((_NPAD, _D), jnp.float32),
            pltpu.SemaphoreType.DMA,
            pltpu.SemaphoreType.DMA,
            pltpu.SemaphoreType.DMA,
            pltpu.SemaphoreType.DMA,
        ],
    )
    def k(table_hbm, gidx_hbm, didx_hbm, z_hbm, out_hbm,
          gidx_v, didx_v, rows_a, rows_b, rows_c, rows_d, z_v, acc,
          sem_a, sem_b, sem_c, sem_d):
        c = lax.axis_index("c")
        s = lax.axis_index("s")
        wid = c * _NS + s

        pltpu.sync_copy(z_hbm, z_v)

        @pl.loop(0, _RPT // 16)
        def _(k2):
            pltpu.sync_copy(z_v, acc.at[pl.ds(s * _RPT + k2 * 16, 16)])

        plsc.subcore_barrier()

        bufs = ((rows_a, sem_a), (rows_b, sem_b), (rows_c, sem_c),
                (rows_d, sem_d))

        @pl.loop(0, _CPT // _CPH)
        def _(ph):
            pltpu.sync_copy(gidx_hbm.at[wid, ph], gidx_v)
            pltpu.sync_copy(didx_hbm.at[wid, ph], didx_v)
            for b, (rows, sem) in enumerate(bufs):
                pltpu.async_copy(table_hbm.at[gidx_v.at[b]], rows, sem)

            @pl.loop(0, _CPH // _NBUF)
            def _(jj):
                j = jj * _NBUF

                def step(off, rows, sem):
                    pltpu.make_async_copy(
                        table_hbm.at[gidx_v.at[j + off]], rows, sem
                    ).wait()
                    pltpu.sync_copy(rows, acc.at[didx_v.at[j + off]], add=True)

                    @pl.when(j + off + _NBUF < _CPH)
                    def _():
                        pltpu.async_copy(
                            table_hbm.at[gidx_v.at[j + off + _NBUF]], rows, sem
                        )

                for b, (rows, sem) in enumerate(bufs):
                    step(b, rows, sem)

        plsc.subcore_barrier()

        pltpu.sync_copy(
            acc.at[pl.ds(s * _RPT, _RPT)],
            out_hbm.at[c, pl.ds(s * _RPT, _RPT)],
        )

    return k(table, gidx, didx, zeros)


def kernel(inputs, edge_index, etypes, W1, loop1, b1, W2, loop2, b2):
    x = inputs
    src = edge_index[0].astype(jnp.int32)
    dst = edge_index[1].astype(jnp.int32)
    et = etypes.astype(jnp.int32)

    pad = _EPAD - src.shape[0]
    pad_g = jnp.arange(pad, dtype=jnp.int32) % (_R * _N)
    pad_d = _N + jnp.arange(pad, dtype=jnp.int32) % (_NPAD - _N)
    idx_shape = (_NW, _CPT // _CPH, _CPH, _CH)
    gidx = jnp.concatenate([et * _N + src, pad_g]).reshape(idx_shape)
    didx = jnp.concatenate([dst, pad_d]).reshape(idx_shape)
    zeros = jnp.zeros((16, _D), jnp.float32)
    b1r = b1.reshape(1, _D)
    b2r = b2.reshape(1, _D)

    xw1, self1 = _transform1(x, W1, loop1, b1r)
    p1 = _sc_aggregate(xw1.reshape(_R * _N, _D), gidx, didx, zeros)
    xw2, self2 = _transform2(p1, self1, W2, loop2, b2r)
    p2 = _sc_aggregate(xw2.reshape(_R * _N, _D), gidx, didx, zeros)
    return _combine(p2, self2)


# speedup vs baseline: 1.0832x; 1.0009x over previous
import functools

import jax
import jax.numpy as jnp
from jax import lax
from jax.experimental import pallas as pl
from jax.experimental.pallas import tpu as pltpu
from jax.experimental.pallas import tpu_sc as plsc

_N = 10000
_E = 320000
_D = 128
_R = 8

_NC = 2
_NS = 16
_NW = _NC * _NS
_CH = 64
_CPT = 160
_CPH = 40
_NBUF = 4
_EPAD = _NW * _CPT * _CH
_NPAD = 10240
_RPT = _NPAD // _NS


def _transform1(x, W, lw, b2d):
    def body(x_ref, w_ref, lw_ref, b_ref, xw_ref, s_ref):
        xb = x_ref[...].astype(jnp.bfloat16)
        wb = w_ref[0].astype(jnp.bfloat16)
        xw_ref[0] = jnp.dot(xb, wb, preferred_element_type=jnp.float32)

        @pl.when(pl.program_id(0) == 0)
        def _():
            lwb = lw_ref[...].astype(jnp.bfloat16)
            s_ref[...] = (
                jnp.dot(xb, lwb, preferred_element_type=jnp.float32)
                + b_ref[...]
            )

    return pl.pallas_call(
        body,
        grid=(_R,),
        in_specs=[
            pl.BlockSpec((_N, _D), lambda r: (0, 0)),
            pl.BlockSpec((1, _D, _D), lambda r: (r, 0, 0)),
            pl.BlockSpec((_D, _D), lambda r: (0, 0)),
            pl.BlockSpec((1, _D), lambda r: (0, 0)),
        ],
        out_specs=[
            pl.BlockSpec((1, _N, _D), lambda r: (r, 0, 0)),
            pl.BlockSpec((_N, _D), lambda r: (0, 0)),
        ],
        out_shape=[
            jax.ShapeDtypeStruct((_R, _N, _D), jnp.float32),
            jax.ShapeDtypeStruct((_N, _D), jnp.float32),
        ],
    )(x, W, lw, b2d)


def _transform2(p, s1, W, lw, b2d):
    def body(p_ref, s1_ref, w_ref, lw_ref, b_ref, xw_ref, s_ref, h_ref):
        @pl.when(pl.program_id(0) == 0)
        def _():
            h_ref[...] = jnp.maximum(
                p_ref[0] + p_ref[1] + s1_ref[...], 0.0
            ).astype(jnp.bfloat16)

        h = h_ref[...]
        wb = w_ref[0].astype(jnp.bfloat16)
        xw_ref[0] = jnp.dot(h, wb, preferred_element_type=jnp.float32)

        @pl.when(pl.program_id(0) == 0)
        def _():
            lwb = lw_ref[...].astype(jnp.bfloat16)
            s_ref[...] = (
                jnp.dot(h, lwb, preferred_element_type=jnp.float32)
                + b_ref[...]
            )

    return pl.pallas_call(
        body,
        grid=(_R,),
        in_specs=[
            pl.BlockSpec((2, _N, _D), lambda r: (0, 0, 0)),
            pl.BlockSpec((_N, _D), lambda r: (0, 0)),
            pl.BlockSpec((1, _D, _D), lambda r: (r, 0, 0)),
            pl.BlockSpec((_D, _D), lambda r: (0, 0)),
            pl.BlockSpec((1, _D), lambda r: (0, 0)),
        ],
        out_specs=[
            pl.BlockSpec((1, _N, _D), lambda r: (r, 0, 0)),
            pl.BlockSpec((_N, _D), lambda r: (0, 0)),
        ],
        out_shape=[
            jax.ShapeDtypeStruct((_R, _N, _D), jnp.float32),
            jax.ShapeDtypeStruct((_N, _D), jnp.float32),
        ],
        scratch_shapes=[pltpu.VMEM((_N, _D), jnp.bfloat16)],
    )(p, s1, W, lw, b2d)


def _combine(p, s2):
    def body(p_ref, s_ref, o_ref):
        o_ref[...] = p_ref[0] + p_ref[1] + s_ref[...]

    return pl.pallas_call(
        body,
        grid=(1,),
        in_specs=[
            pl.BlockSpec((2, _N, _D), lambda i: (0, 0, 0)),
            pl.BlockSpec((_N, _D), lambda i: (0, 0)),
        ],
        out_specs=pl.BlockSpec((_N, _D), lambda i: (0, 0)),
        out_shape=jax.ShapeDtypeStruct((_N, _D), jnp.float32),
    )(p, s2)


def _sc_aggregate(table, gidx, didx, zeros):
    mesh = plsc.VectorSubcoreMesh(core_axis_name="c", subcore_axis_name="s")

    @functools.partial(
        pl.kernel,
        mesh=mesh,
        out_type=jax.ShapeDtypeStruct((_NC, _NPAD, _D), jnp.float32),
        scratch_types=[
            pltpu.VMEM((_CPH, _CH), jnp.int32),
            pltpu.VMEM((_CPH, _CH), jnp.int32),
            pltpu.VMEM((_CH, _D), jnp.float32),
            pltpu.VMEM((_CH, _D), jnp.float32),
            pltpu.VMEM((_CH, _D), jnp.float32),
            pltpu.VMEM((_CH, _D), jnp.float32),
            pltpu.VMEM((16, _D), jnp.float32),
            pltpu.VMEM_SHARED((_NPAD, _D), jnp.float32),
            pltpu.SemaphoreType.DMA,
            pltpu.SemaphoreType.DMA,
            pltpu.SemaphoreType.DMA,
            pltpu.SemaphoreType.DMA,
        ],
    )
    def k(table_hbm, gidx_hbm, didx_hbm, z_hbm, out_hbm,
          gidx_v, didx_v, rows_a, rows_b, rows_c, rows_d, z_v, acc,
          sem_a, sem_b, sem_c, sem_d):
        c = lax.axis_index("c")
        s = lax.axis_index("s")
        wid = c * _NS + s

        pltpu.sync_copy(z_hbm, z_v)

        @pl.loop(0, _RPT // 16)
        def _(k2):
            pltpu.sync_copy(z_v, acc.at[pl.ds(s * _RPT + k2 * 16, 16)])

        plsc.subcore_barrier()

        bufs = ((rows_a, sem_a), (rows_b, sem_b), (rows_c, sem_c),
                (rows_d, sem_d))

        @pl.loop(0, _CPT // _CPH)
        def _(ph):
            pltpu.sync_copy(gidx_hbm.at[wid, ph], gidx_v)
            pltpu.sync_copy(didx_hbm.at[wid, ph], didx_v)
            for b, (rows, sem) in enumerate(bufs):
                pltpu.async_copy(table_hbm.at[gidx_v.at[b]], rows, sem)

            @pl.loop(0, _CPH // _NBUF)
            def _(jj):
                j = jj * _NBUF

                def step(off, rows, sem):
                    pltpu.make_async_copy(
                        table_hbm.at[gidx_v.at[j + off]], rows, sem
                    ).wait()
                    pltpu.sync_copy(rows, acc.at[didx_v.at[j + off]], add=True)

                    @pl.when(j + off + _NBUF < _CPH)
                    def _():
                        pltpu.async_copy(
                            table_hbm.at[gidx_v.at[j + off + _NBUF]], rows, sem
                        )

                for b, (rows, sem) in enumerate(bufs):
                    step(b, rows, sem)

        plsc.subcore_barrier()

        pltpu.sync_copy(
            acc.at[pl.ds(s * _RPT, _RPT)],
            out_hbm.at[c, pl.ds(s * _RPT, _RPT)],
        )

    return k(table, gidx, didx, zeros)


def kernel(inputs, edge_index, etypes, W1, loop1, b1, W2, loop2, b2):
    x = inputs
    src = edge_index[0].astype(jnp.int32)
    dst = edge_index[1].astype(jnp.int32)
    et = etypes.astype(jnp.int32)

    pad = _EPAD - src.shape[0]
    pad_g = jnp.arange(pad, dtype=jnp.int32) % (_R * _N)
    pad_d = _N + jnp.arange(pad, dtype=jnp.int32) % (_NPAD - _N)
    idx_shape = (_NW, _CPT // _CPH, _CPH, _CH)
    gidx = jnp.concatenate([et * _N + src, pad_g]).reshape(idx_shape)
    didx = jnp.concatenate([dst, pad_d]).reshape(idx_shape)
    zeros = jnp.zeros((16, _D), jnp.float32)
    b1r = b1.reshape(1, _D)
    b2r = b2.reshape(1, _D)

    xw1, self1 = _transform1(x, W1, loop1, b1r)
    p1 = _sc_aggregate(xw1.reshape(_R * _N, _D), gidx, didx, zeros)
    xw2, self2 = _transform2(p1, self1, W2, loop2, b2r)
    p2 = _sc_aggregate(xw2.reshape(_R * _N, _D), gidx, didx, zeros)
    return _combine(p2, self2)
